# baseline (speedup 1.0000x reference)
import functools

import jax
import jax.numpy as jnp
import numpy as np
from jax import lax
from jax.experimental import pallas as pl
from jax.experimental.pallas import tpu as pltpu
from jax.experimental.pallas import tpu_sc as plsc

N_NODES = 10000
DEG = 32
D = 128

NC = 2
NS = 16
NW = NC * NS

P = 320
N_PAD = NW * P
C = 4
E = C * DEG
CH = P // C
GROUPS = D // 32


NBUF = 2


ROWS_PER_TILE = N_PAD // NS


FL = 8


def _sc_body(neigh_ref, x_ref, out_ref, idx_v, x_sh, rows_0, rows_1,
             out_v, sem_x, sem_0, sem_1, osem):
    rows = (rows_0, rows_1)
    sems = (sem_0, sem_1)
    sid = lax.axis_index("s")
    wid = sid * NC + lax.axis_index("c")
    base_n = wid * P

    stage = pltpu.async_copy(
        x_ref.at[pl.ds(sid * ROWS_PER_TILE, ROWS_PER_TILE)],
        x_sh.at[pl.ds(sid * ROWS_PER_TILE, ROWS_PER_TILE)], sem_x)

    pltpu.sync_copy(neigh_ref.at[pl.ds(wid * CH, CH)], idx_v)
    stage.wait()
    plsc.subcore_barrier()

    for b in range(NBUF):
        pltpu.async_copy(x_sh.at[idx_v.at[b]], rows[b], sems[b])

    hi_mask = jnp.full((16,), -65536, dtype=jnp.int32)
    shift16 = jnp.full((16,), 16, dtype=jnp.int32)

    def compute(rows_ref, ro):
        def node_body(n, carry):
            r0 = n * DEG
            lo = [None] * GROUPS
            hi = [None] * GROUPS
            for j in range(DEG):
                for h in range(GROUPS):
                    w = rows_ref[r0 + j, pl.ds(16 * h, 16)]
                    l = lax.bitcast_convert_type(lax.shift_left(w, shift16), jnp.float32)
                    u = lax.bitcast_convert_type(lax.bitwise_and(w, hi_mask),
                                     jnp.float32)
                    if j == 0:
                        lo[h], hi[h] = l, u
                    else:
                        lo[h], hi[h] = lo[h] + l, hi[h] + u
            for h in range(GROUPS):
                out_v[ro + n, pl.ds(32 * h, 16)] = lo[h]
                out_v[ro + n, pl.ds(32 * h + 16, 16)] = hi[h]
            return carry
        lax.fori_loop(0, C, node_body, 0)

    def ring_body(q, carry):
        for b in range(NBUF):
            c = NBUF * q + b
            rem = lax.rem(c, FL)

            pltpu.make_async_copy(x_sh.at[idx_v.at[c]], rows[b],
                                  sems[b]).wait()

            @pl.when(jnp.logical_and(rem == 0, c > 0))
            def _():
                pltpu.make_async_copy(
                    out_v, out_ref.at[pl.ds(base_n, FL * C)], osem).wait()

            compute(rows[b], rem * C)

            @pl.when(c + NBUF < CH)
            def _():
                pltpu.async_copy(x_sh.at[idx_v.at[c + NBUF]], rows[b],
                                 sems[b])

            @pl.when(rem == FL - 1)
            def _():
                gi = lax.div(c, FL)
                pltpu.async_copy(
                    out_v, out_ref.at[pl.ds(base_n + gi * FL * C, FL * C)],
                    osem)

        return carry

    lax.fori_loop(0, CH // NBUF, ring_body, 0)

    pltpu.make_async_copy(out_v, out_ref.at[pl.ds(base_n, FL * C)],
                          osem).wait()


_sc_gather_sum = pl.kernel(
    _sc_body,
    out_type=jax.ShapeDtypeStruct((N_PAD, D), jnp.float32),
    mesh=plsc.VectorSubcoreMesh(core_axis_name="c", subcore_axis_name="s"),
    scratch_types=[
        pltpu.VMEM((CH, E), jnp.int32),
        pltpu.VMEM_SHARED((N_PAD, D // 2), jnp.int32),
        pltpu.VMEM((E, D // 2), jnp.int32),
        pltpu.VMEM((E, D // 2), jnp.int32),
        pltpu.VMEM((FL * C, D), jnp.float32),
        pltpu.SemaphoreType.DMA,
        pltpu.SemaphoreType.DMA,
        pltpu.SemaphoreType.DMA,
        pltpu.SemaphoreType.DMA,
    ],
)


def _tc_body(x_ref, a_ref, w1_ref, w2_ref, b_ref, o_ref):
    o_ref[...] = (
        jnp.dot(x_ref[...], w1_ref[...], preferred_element_type=jnp.float32)
        + jnp.dot(a_ref[...], w2_ref[...], preferred_element_type=jnp.float32)
        + b_ref[...]
    )


_R = 2000

_DEINT_PERM = np.concatenate(
    [np.concatenate([np.arange(32 * h, 32 * h + 32, 2),
                     np.arange(32 * h + 1, 32 * h + 32, 2)])
     for h in range(GROUPS)])


@jax.jit
def _run(neighbours, x, W1, b1, W2, b2):
    neigh = neighbours.astype(jnp.int32).reshape(-1)
    neigh = jnp.pad(neigh, (0, (N_PAD - N_NODES) * DEG))
    neigh = neigh.reshape(N_PAD * DEG // E, E)

    x_bf = jnp.pad(x, ((0, N_PAD - N_NODES), (0, 0))).astype(jnp.bfloat16)
    u = lax.bitcast_convert_type(x_bf, jnp.uint16).astype(jnp.int32)
    x_pk = u[:, 0::2] | (u[:, 1::2] << 16)
    aggr = _sc_gather_sum(neigh, x_pk)

    W2p = W2[_DEINT_PERM, :]
    bsum = (b1 + b2).reshape(1, D)
    out = pl.pallas_call(
        _tc_body,
        grid=(N_NODES // _R,),
        in_specs=[
            pl.BlockSpec((_R, D), lambda i: (i, 0)),
            pl.BlockSpec((_R, D), lambda i: (i, 0)),
            pl.BlockSpec((D, D), lambda i: (0, 0)),
            pl.BlockSpec((D, D), lambda i: (0, 0)),
            pl.BlockSpec((1, D), lambda i: (0, 0)),
        ],
        out_specs=pl.BlockSpec((_R, D), lambda i: (i, 0)),
        out_shape=jax.ShapeDtypeStruct((N_NODES, D), jnp.float32),
    )(x, aggr, W1, W2p, bsum)
    return out


def kernel(neighbours, shape_features, W1, b1, W2, b2):
    return _run(neighbours, shape_features, W1, b1, W2, b2)


# speedup vs baseline: 3.1796x; 3.1796x over previous
import functools

import jax
import jax.numpy as jnp
import numpy as np
from jax import lax
from jax.experimental import pallas as pl
from jax.experimental.pallas import tpu as pltpu
from jax.experimental.pallas import tpu_sc as plsc

N_NODES = 10000
DEG = 32
D = 128

NC = 2
NS = 16
NW = NC * NS

P = 320
N_PAD = NW * P
C = 4
E = C * DEG
CH = P // C
GROUPS = D // 32


NBUF = 2


ROWS_PER_TILE = N_PAD // NS


FL = 8


def _sc_body(neigh_ref, x_ref, out_ref, idx_v, x_sh, rows_0, rows_1,
             out_v, sem_x, sem_0, sem_1, osem):
    rows = (rows_0, rows_1)
    sems = (sem_0, sem_1)
    sid = lax.axis_index("s")
    wid = sid * NC + lax.axis_index("c")
    base_n = wid * P

    stage = pltpu.async_copy(
        x_ref.at[pl.ds(sid * ROWS_PER_TILE, ROWS_PER_TILE)],
        x_sh.at[pl.ds(sid * ROWS_PER_TILE, ROWS_PER_TILE)], sem_x)

    pltpu.sync_copy(neigh_ref.at[pl.ds(wid * CH, CH)], idx_v)
    stage.wait()
    plsc.subcore_barrier()

    for b in range(NBUF):
        pltpu.async_copy(x_sh.at[idx_v.at[b]], rows[b], sems[b])

    hi_mask = jnp.full((16,), -65536, dtype=jnp.int32)
    shift16 = jnp.full((16,), 16, dtype=jnp.int32)

    def compute(rows_ref, ro):
        def node_body(n, carry):
            r0 = n * DEG
            lo = [None] * GROUPS
            hi = [None] * GROUPS
            for j in range(DEG):
                for h in range(GROUPS):
                    w = rows_ref[r0 + j, pl.ds(16 * h, 16)]
                    l = lax.bitcast_convert_type(lax.shift_left(w, shift16), jnp.float32)
                    u = lax.bitcast_convert_type(lax.bitwise_and(w, hi_mask),
                                     jnp.float32)
                    if j == 0:
                        lo[h], hi[h] = l, u
                    else:
                        lo[h], hi[h] = lo[h] + l, hi[h] + u
            for h in range(GROUPS):
                out_v[ro + n, pl.ds(16 * h, 16)] = lo[h]
                out_v[ro + n, pl.ds(D // 2 + 16 * h, 16)] = hi[h]
            return carry
        lax.fori_loop(0, C, node_body, 0)

    def ring_body(q, carry):
        for b in range(NBUF):
            c = NBUF * q + b
            rem = lax.rem(c, FL)

            pltpu.make_async_copy(x_sh.at[idx_v.at[c]], rows[b],
                                  sems[b]).wait()

            @pl.when(jnp.logical_and(rem == 0, c > 0))
            def _():
                pltpu.make_async_copy(
                    out_v, out_ref.at[pl.ds(base_n, FL * C)], osem).wait()

            compute(rows[b], rem * C)

            @pl.when(c + NBUF < CH)
            def _():
                pltpu.async_copy(x_sh.at[idx_v.at[c + NBUF]], rows[b],
                                 sems[b])

            @pl.when(rem == FL - 1)
            def _():
                gi = lax.div(c, FL)
                pltpu.async_copy(
                    out_v, out_ref.at[pl.ds(base_n + gi * FL * C, FL * C)],
                    osem)

        return carry

    lax.fori_loop(0, CH // NBUF, ring_body, 0)

    pltpu.make_async_copy(out_v, out_ref.at[pl.ds(base_n, FL * C)],
                          osem).wait()


_sc_gather_sum = pl.kernel(
    _sc_body,
    out_type=jax.ShapeDtypeStruct((N_PAD, D), jnp.float32),
    mesh=plsc.VectorSubcoreMesh(core_axis_name="c", subcore_axis_name="s"),
    scratch_types=[
        pltpu.VMEM((CH, E), jnp.int32),
        pltpu.VMEM_SHARED((N_PAD, D // 2), jnp.int32),
        pltpu.VMEM((E, D // 2), jnp.int32),
        pltpu.VMEM((E, D // 2), jnp.int32),
        pltpu.VMEM((FL * C, D), jnp.float32),
        pltpu.SemaphoreType.DMA,
        pltpu.SemaphoreType.DMA,
        pltpu.SemaphoreType.DMA,
        pltpu.SemaphoreType.DMA,
    ],
)


def _pack_body(x_ref, o_ref):
    u = lax.bitcast_convert_type(x_ref[...].astype(jnp.bfloat16),
                                 jnp.uint16).astype(jnp.int32)
    o_ref[...] = u[:, :D // 2] | (u[:, D // 2:] << 16)


def _tc_body(x_ref, a_ref, w1_ref, w2_ref, b_ref, o_ref):
    o_ref[...] = (
        jnp.dot(x_ref[...], w1_ref[...], preferred_element_type=jnp.float32)
        + jnp.dot(a_ref[...], w2_ref[...], preferred_element_type=jnp.float32)
        + b_ref[...]
    )


_R = 2000


@jax.jit
def _run(neighbours, x, W1, b1, W2, b2):
    neigh = neighbours.astype(jnp.int32).reshape(-1)
    neigh = jnp.pad(neigh, (0, (N_PAD - N_NODES) * DEG))
    neigh = neigh.reshape(N_PAD * DEG // E, E)

    x_pad = jnp.pad(x, ((0, N_PAD - N_NODES), (0, 0)))
    x_pk = pl.pallas_call(
        _pack_body,
        grid=(N_PAD // 2048,),
        in_specs=[pl.BlockSpec((2048, D), lambda i: (i, 0))],
        out_specs=pl.BlockSpec((2048, D // 2), lambda i: (i, 0)),
        out_shape=jax.ShapeDtypeStruct((N_PAD, D // 2), jnp.int32),
    )(x_pad)
    aggr = _sc_gather_sum(neigh, x_pk)

    bsum = (b1 + b2).reshape(1, D)
    out = pl.pallas_call(
        _tc_body,
        grid=(N_NODES // _R,),
        in_specs=[
            pl.BlockSpec((_R, D), lambda i: (i, 0)),
            pl.BlockSpec((_R, D), lambda i: (i, 0)),
            pl.BlockSpec((D, D), lambda i: (0, 0)),
            pl.BlockSpec((D, D), lambda i: (0, 0)),
            pl.BlockSpec((1, D), lambda i: (0, 0)),
        ],
        out_specs=pl.BlockSpec((_R, D), lambda i: (i, 0)),
        out_shape=jax.ShapeDtypeStruct((N_NODES, D), jnp.float32),
    )(x, aggr, W1, W2, bsum)
    return out


def kernel(neighbours, shape_features, W1, b1, W2, b2):
    return _run(neighbours, shape_features, W1, b1, W2, b2)
